# baseline (speedup 1.0000x reference)
import dataclasses
import functools

import jax
import jax.numpy as jnp
from jax import lax
from jax.experimental import pallas as pl
from jax.experimental.pallas import tpu as pltpu
from jax.experimental.pallas import tpu_sc as plsc

N = 10000
T = 10240
E_RAW = 160000
E_TOT = E_RAW + N
K = 128
NTILE = 16
EPT = 10752
E_PAD = EPT * NTILE
NCH = EPT // K
H = 256
HH = 128
B = 128
NOUT = 256
DUMMY = N
BL = T // 8
CS = 8192
CG = E_PAD // CS

_mesh = plsc.VectorSubcoreMesh(core_axis_name="c", subcore_axis_name="s")

_sc_cp = pltpu.CompilerParams()
if "needs_layout_passes" in pltpu.CompilerParams.__dataclass_fields__:
    _sc_cp = dataclasses.replace(_sc_cp, needs_layout_passes=False)
if "use_tc_tiling_on_sc" in pltpu.CompilerParams.__dataclass_fields__:
    _sc_cp = dataclasses.replace(_sc_cp, use_tc_tiling_on_sc=False)


@functools.partial(
    pl.kernel,
    mesh=_mesh,
    out_type=jax.ShapeDtypeStruct((2 * E_PAD, 16), jnp.float32),
    scratch_types=[
        pltpu.VMEM((K,), jnp.int32),
        pltpu.VMEM((K,), jnp.int32),
        pltpu.VMEM((K,), jnp.int32),
        pltpu.VMEM((K, HH), jnp.float32),
        pltpu.VMEM((K, HH), jnp.float32),
        pltpu.VMEM((K, 16), jnp.float32),
        pltpu.VMEM((HH,), jnp.float32),
    ],
)
def _sc_logits(xl_hbm, xr_hbm, src_hbm, dst_hbm, att_hbm, s_hbm,
               sraw, sidx, didx, xlv, xrv, sv, attv):
    c = lax.axis_index("c")
    sub = lax.axis_index("s")
    off = c * T
    pltpu.sync_copy(att_hbm.at[pl.ds(c * HH, HH)], attv)
    att_regs = [attv[pl.ds(16 * j, 16)] for j in range(8)]
    tbase = sub * EPT

    @pl.loop(0, NCH)
    def _chunk(ch):
        base = tbase + ch * K
        pltpu.sync_copy(src_hbm.at[pl.ds(base, K)], sraw)
        pltpu.sync_copy(dst_hbm.at[pl.ds(base, K)], didx)
        for g in range(K // 16):
            sl = pl.ds(16 * g, 16)
            sidx[sl] = sraw[sl] + off
            didx[sl] = didx[sl] + off
        pltpu.sync_copy(xl_hbm.at[sidx], xlv)
        pltpu.sync_copy(xr_hbm.at[didx], xrv)

        @pl.loop(0, K)
        def _edge(e):
            rl = xlv.at[e]
            rr = xrv.at[e]
            tot = None
            for j in range(8):
                sl = pl.ds(16 * j, 16)
                a = rl[sl] + rr[sl]
                lr = jnp.maximum(a, a * jnp.float32(0.2))
                t = lr * att_regs[j]
                tot = t if tot is None else tot + t
            sv.at[e][...] = tot

        pltpu.sync_copy(sv, s_hbm.at[pl.ds(c * E_PAD + base, K)])


KB = 128


@functools.partial(
    pl.kernel,
    mesh=_mesh,
    out_type=[
        jax.ShapeDtypeStruct((32 * T * 8,), jnp.float32),
        jax.ShapeDtypeStruct((T,), jnp.float32),
    ],
    scratch_types=[
        pltpu.VMEM((KB,), jnp.int32),
        pltpu.VMEM((KB,), jnp.int32),
        pltpu.VMEM((KB,), jnp.int32),
        pltpu.VMEM((KB,), jnp.float32),
        pltpu.VMEM((KB, 16), jnp.float32),
        pltpu.VMEM((T * 8,), jnp.float32),
        pltpu.VMEM((T,), jnp.float32),
    ],
    compiler_params=_sc_cp,
)
def _sc_aggregate(xl_hbm, src_hbm, dst_hbm, p_hbm, acc_hbm, den_hbm,
                  sraw, sidx, didx, pv, xg, slab, dslab):
    c = lax.axis_index("c")
    sub = lax.axis_index("s")
    g = c * 16 + sub
    q = sub // 2
    o = sub % 2
    rbase = c * T * 8 + q
    iota16 = lax.broadcasted_iota(jnp.int32, (16,), 0)
    lane0 = iota16 == 0
    halfmask = (iota16 // 8) == o
    addr_lo = iota16 % 8
    zeros16 = jnp.zeros((16,), jnp.float32)

    @pl.loop(0, T * 8 // 16)
    def _z(r):
        slab[pl.ds(r * 16, 16)] = zeros16

    @pl.loop(0, T // 16)
    def _zd(r):
        dslab[pl.ds(r * 16, 16)] = zeros16

    @pl.loop(0, E_PAD // KB)
    def _chunk(ch):
        base = ch * KB
        pltpu.sync_copy(src_hbm.at[pl.ds(base, KB)], sraw)
        pltpu.sync_copy(dst_hbm.at[pl.ds(base, KB)], didx)
        pltpu.sync_copy(p_hbm.at[pl.ds(base, KB)], pv)
        for gg in range(KB // 16):
            sl = pl.ds(16 * gg, 16)
            sidx[sl] = sraw[sl] * 8 + rbase
        pltpu.sync_copy(xl_hbm.at[sidx], xg)

        @pl.loop(0, KB // 16)
        def _grp(gg):
            dg = didx[pl.ds(16 * gg, 16)]
            pg = pv[pl.ds(16 * gg, 16)]
            for i in range(16):
                e = 16 * gg + i
                d = dg[i]
                pb = jnp.full((16,), pg[i], jnp.float32)
                val = xg.at[e][...] * pb
                addr = jnp.full((16,), d * 8, jnp.int32) + addr_lo
                plsc.addupdate_scatter(slab, [addr], val, mask=halfmask)
                plsc.addupdate_scatter(
                    dslab, [jnp.full((16,), d, jnp.int32)],
                    pb, mask=lane0)

    pltpu.sync_copy(slab, acc_hbm.at[pl.ds(g * T * 8, T * 8)])
    pltpu.sync_copy(dslab, den_hbm)


def _prep_body(h_ref, wl_ref, bl_ref, wr_ref, br_ref, xl_ref, xr_ref):
    hb = h_ref[...]
    xl_ref[0] = (jnp.dot(hb, wl_ref[0], preferred_element_type=jnp.float32)
                 + bl_ref[0])
    xr_ref[0] = (jnp.dot(hb, wr_ref[0], preferred_element_type=jnp.float32)
                 + br_ref[0])


def _tc_prep(h_pad, Wl, bl, Wr, br):
    return pl.pallas_call(
        _prep_body,
        grid=(8, 2),
        in_specs=[
            pl.BlockSpec((BL, H), lambda i, c: (i, 0)),
            pl.BlockSpec((1, H, HH), lambda i, c: (c, 0, 0)),
            pl.BlockSpec((1, 1, HH), lambda i, c: (c, 0, 0)),
            pl.BlockSpec((1, H, HH), lambda i, c: (c, 0, 0)),
            pl.BlockSpec((1, 1, HH), lambda i, c: (c, 0, 0)),
        ],
        out_specs=[
            pl.BlockSpec((1, BL, HH), lambda i, c: (c, i, 0)),
            pl.BlockSpec((1, BL, HH), lambda i, c: (c, i, 0)),
        ],
        out_shape=[
            jax.ShapeDtypeStruct((2, T, HH), jnp.float32),
            jax.ShapeDtypeStruct((2, T, HH), jnp.float32),
        ],
    )(h_pad, Wl.reshape(H, 2, HH).transpose(1, 0, 2),
      bl.reshape(2, 1, HH),
      Wr.reshape(H, 2, HH).transpose(1, 0, 2),
      br.reshape(2, 1, HH))


def _combine_body(s_ref, p_ref):
    s = s_ref[0, 0] + s_ref[1, 0]
    p_ref[0, 0] = jnp.exp(jnp.sum(s, axis=1))


def _tc_combine(S):
    return pl.pallas_call(
        _combine_body,
        grid=(CG,),
        in_specs=[pl.BlockSpec((2, 1, CS, 16), lambda i: (0, i, 0, 0))],
        out_specs=pl.BlockSpec((1, 1, CS), lambda i: (i, 0, 0)),
        out_shape=jax.ShapeDtypeStruct((CG, 1, CS), jnp.float32),
    )(S.reshape(2, CG, CS, 16))


def _post_body(acc_ref, den_ref, b_ref, g_ref, lb_ref, h_ref):
    a = acc_ref[...]
    d = den_ref[...]
    h = a / (d + jnp.float32(1e-16)) + b_ref[...]
    h = jnp.maximum(h, jnp.float32(0.0))
    mu = jnp.mean(h, axis=1, keepdims=True)
    diff = h - mu
    var = jnp.mean(diff * diff, axis=1, keepdims=True)
    h = diff * lax.rsqrt(var + jnp.float32(1e-5)) * g_ref[...] + lb_ref[...]
    row = (pl.program_id(0) * BL
           + lax.broadcasted_iota(jnp.int32, (BL, 1), 0))
    h_ref[...] = jnp.where(row < N, h, jnp.float32(0.0))


def _tc_post(acc, den, b, ln_g, ln_b):
    return pl.pallas_call(
        _post_body,
        grid=(8,),
        in_specs=[
            pl.BlockSpec((BL, H), lambda i: (i, 0)),
            pl.BlockSpec((BL, 1), lambda i: (i, 0)),
            pl.BlockSpec((1, H), lambda i: (0, 0)),
            pl.BlockSpec((1, H), lambda i: (0, 0)),
            pl.BlockSpec((1, H), lambda i: (0, 0)),
        ],
        out_specs=pl.BlockSpec((BL, H), lambda i: (i, 0)),
        out_shape=jax.ShapeDtypeStruct((T, H), jnp.float32),
    )(acc, den, b.reshape(1, H), ln_g.reshape(1, H), ln_b.reshape(1, H))


def _pool_body(h_ref, bt_ref, lcw_ref, lcb_ref, o_ref):
    hb = h_ref[...]
    m = (bt_ref[...] == lax.broadcasted_iota(jnp.int32, (T, B), 1)
         ).astype(jnp.float32)
    sums = lax.dot_general(m, hb, (((0,), (0,)), ((), ())),
                           preferred_element_type=jnp.float32)
    cnt = jnp.sum(m, axis=0)
    g = sums / jnp.maximum(cnt, jnp.float32(1.0))[:, None]
    o_ref[...] = jnp.maximum(
        jnp.dot(g, lcw_ref[...], preferred_element_type=jnp.float32)
        + lcb_ref[...], jnp.float32(0.0))


def _tc_pool(h_pad, batch_pad, lcW, lcb):
    return pl.pallas_call(
        _pool_body,
        out_shape=jax.ShapeDtypeStruct((B, NOUT), jnp.float32),
    )(h_pad, batch_pad, lcW, lcb.reshape(1, NOUT))


def kernel(x, edge_index, batch, Wl0, bl0, Wr0, br0, att0, b0,
           Wl1, bl1, Wr1, br1, att1, b1, Wl2, bl2, Wr2, br2, att2, b2,
           ln_g, ln_b, lcW, lcb):
    loops = jnp.arange(N, dtype=jnp.int32)
    pad = jnp.full((E_PAD - E_TOT,), DUMMY, jnp.int32)
    src = jnp.concatenate([edge_index[0].astype(jnp.int32), loops, pad])
    dst = jnp.concatenate([edge_index[1].astype(jnp.int32), loops, pad])
    h = jnp.pad(x, ((0, T - N), (0, 0)))
    batch_pad = jnp.pad(batch.astype(jnp.int32), (0, T - N),
                        constant_values=2 ** 30).reshape(T, 1)
    for (Wl, bl, Wr, br, att, b) in (
            (Wl0, bl0, Wr0, br0, att0, b0),
            (Wl1, bl1, Wr1, br1, att1, b1),
            (Wl2, bl2, Wr2, br2, att2, b2)):
        XL, XR = _tc_prep(h, Wl, bl, Wr, br)
        XLf = XL.reshape(2 * T, HH)
        XRf = XR.reshape(2 * T, HH)
        S = _sc_logits(XLf, XRf, src, dst, att)
        p = _tc_combine(S).reshape(E_PAD)
        accf, den = _sc_aggregate(XLf.reshape(2 * T * 8, 16), src, dst, p)
        acc = accf.reshape(32, T, 8).transpose(1, 0, 2).reshape(T, H)
        h = _tc_post(acc, den.reshape(T, 1), b, ln_g, ln_b)
    return _tc_pool(h, batch_pad, lcW, lcb)
